# baseline (device time: 48946 ns/iter reference)
import jax
import jax.numpy as jnp
from jax import lax
from jax.experimental import pallas as pl
from jax.experimental.pallas import tpu as pltpu

N_DEV = 4
B_LOC = 2
SQ = 128
SKV = 128
HQ = 16
HQ_GRP = 4
DH = 64
D_MODEL = 512
D_QKV = 256


def kernel(x, Wq, K_ext, V_ext, Wo):
    my = lax.axis_index("i")
    K_own = jnp.transpose(
        lax.dynamic_slice_in_dim(K_ext, B_LOC * my, B_LOC, axis=0), (0, 2, 1, 3)
    )
    V_own = jnp.transpose(
        lax.dynamic_slice_in_dim(V_ext, B_LOC * my, B_LOC, axis=0), (0, 2, 1, 3)
    )

    def body(x_ref, wq_ref, k_ref, v_ref, wo_ref, out_ref,
             wq_comm, wo_comm, wq_send, wq_recv, wo_send, wo_recv):
        my_pos = lax.axis_index("i")
        left = lax.rem(my_pos + N_DEV - 1, N_DEV)
        right = lax.rem(my_pos + 1, N_DEV)

        wq_comm[0] = wq_ref[...]
        wo_comm[0] = wo_ref[...]

        barrier_sem = pltpu.get_barrier_semaphore()
        for nbr in (left, right):
            pl.semaphore_signal(barrier_sem, inc=1, device_id=(nbr,),
                                device_id_type=pl.DeviceIdType.MESH)
        pl.semaphore_wait(barrier_sem, 2)

        def compute(h):
            g = lax.rem(my_pos + N_DEV - h, N_DEV)
            wq = wq_comm[h]
            wo = wo_comm[h]
            for b in range(B_LOC):
                qb = jnp.dot(x_ref[b], wq, preferred_element_type=jnp.float32)
                ctxs = []
                for hh in range(HQ_GRP):
                    head = g * HQ_GRP + hh
                    q = qb[:, hh * DH:(hh + 1) * DH]
                    k = k_ref[b, head]
                    v = v_ref[b, head]
                    s = lax.dot_general(
                        q, k, (((1,), (1,)), ((), ())),
                        preferred_element_type=jnp.float32) * 0.125
                    m = jnp.max(s, axis=1, keepdims=True)
                    p = jnp.exp(s - m)
                    w = p / jnp.sum(p, axis=1, keepdims=True)
                    ctxs.append(jnp.dot(w, v, preferred_element_type=jnp.float32))
                ctx = jnp.concatenate(ctxs, axis=1)
                contrib = jnp.dot(ctx, wo, preferred_element_type=jnp.float32)
                if h == 0:
                    out_ref[b] = contrib
                else:
                    out_ref[b] = out_ref[b] + contrib

        sends = []
        for h in range(N_DEV):
            if h < N_DEV - 1:
                wq_r = pltpu.make_async_remote_copy(
                    src_ref=wq_comm.at[h], dst_ref=wq_comm.at[h + 1],
                    send_sem=wq_send.at[h], recv_sem=wq_recv.at[h],
                    device_id=(right,), device_id_type=pl.DeviceIdType.MESH)
                wo_r = pltpu.make_async_remote_copy(
                    src_ref=wo_comm.at[h], dst_ref=wo_comm.at[h + 1],
                    send_sem=wo_send.at[h], recv_sem=wo_recv.at[h],
                    device_id=(right,), device_id_type=pl.DeviceIdType.MESH)
                wq_r.start()
                wo_r.start()
                sends += [wq_r, wo_r]
            compute(h)
            if h < N_DEV - 1:
                wq_r.wait_recv()
                wo_r.wait_recv()
        for r in sends:
            r.wait_send()

    return pl.pallas_call(
        body,
        out_shape=jax.ShapeDtypeStruct((B_LOC, SQ, D_MODEL), jnp.float32),
        in_specs=[pl.BlockSpec(memory_space=pltpu.VMEM)] * 5,
        out_specs=pl.BlockSpec(memory_space=pltpu.VMEM),
        scratch_shapes=[
            pltpu.VMEM((N_DEV, D_MODEL, D_QKV), jnp.float32),
            pltpu.VMEM((N_DEV, D_QKV, D_MODEL), jnp.float32),
            pltpu.SemaphoreType.DMA((N_DEV - 1,)),
            pltpu.SemaphoreType.DMA((N_DEV - 1,)),
            pltpu.SemaphoreType.DMA((N_DEV - 1,)),
            pltpu.SemaphoreType.DMA((N_DEV - 1,)),
        ],
        compiler_params=pltpu.CompilerParams(collective_id=0),
    )(x, Wq, K_own, V_own, Wo)


# device time: 34164 ns/iter; 1.4327x vs baseline; 1.4327x over previous
import jax
import jax.numpy as jnp
from jax import lax
from jax.experimental import pallas as pl
from jax.experimental.pallas import tpu as pltpu

N_DEV = 4
B_LOC = 2
SQ = 128
SKV = 128
HQ = 16
HQ_GRP = 4
DH = 64
D_MODEL = 512
D_QKV = 256


def kernel(x, Wq, K_ext, V_ext, Wo):
    my = lax.axis_index("i")
    K_own = jnp.transpose(
        lax.dynamic_slice_in_dim(K_ext, B_LOC * my, B_LOC, axis=0), (0, 2, 1, 3)
    )
    V_own = jnp.transpose(
        lax.dynamic_slice_in_dim(V_ext, B_LOC * my, B_LOC, axis=0), (0, 2, 1, 3)
    )

    def body(x_ref, wq_ref, k_ref, v_ref, wo_ref, out_ref,
             wq_comm, wo_comm, wq_send, wq_recv, wo_send, wo_recv):
        my_pos = lax.axis_index("i")
        left = lax.rem(my_pos + N_DEV - 1, N_DEV)
        right = lax.rem(my_pos + 1, N_DEV)
        opp = lax.rem(my_pos + 2, N_DEV)

        barrier_sem = pltpu.get_barrier_semaphore()
        for nbr in (left, right, opp):
            pl.semaphore_signal(barrier_sem, inc=1, device_id=(nbr,),
                                device_id_type=pl.DeviceIdType.MESH)
        pl.semaphore_wait(barrier_sem, 3)

        rdmas = []
        for slot, tgt in ((0, right), (1, left), (2, opp)):
            for src, comm, ssem, rsem in (
                (wq_ref, wq_comm, wq_send, wq_recv),
                (wo_ref, wo_comm, wo_send, wo_recv),
            ):
                r = pltpu.make_async_remote_copy(
                    src_ref=src, dst_ref=comm.at[slot],
                    send_sem=ssem.at[slot], recv_sem=rsem.at[slot],
                    device_id=(tgt,), device_id_type=pl.DeviceIdType.MESH)
                r.start()
                rdmas.append(r)

        def compute(g, wq, wo, first):
            xm = x_ref[...].reshape(B_LOC * SQ, D_MODEL)
            qm = jnp.dot(xm, wq, preferred_element_type=jnp.float32)
            ctxs = []
            for b in range(B_LOC):
                qb = qm[b * SQ:(b + 1) * SQ]
                for hh in range(HQ_GRP):
                    head = g * HQ_GRP + hh
                    q = qb[:, hh * DH:(hh + 1) * DH]
                    k = k_ref[b, head]
                    v = v_ref[b, head]
                    s = lax.dot_general(
                        q, k, (((1,), (1,)), ((), ())),
                        preferred_element_type=jnp.float32) * 0.125
                    m = jnp.max(s, axis=1, keepdims=True)
                    p = jnp.exp(s - m)
                    w = p / jnp.sum(p, axis=1, keepdims=True)
                    ctxs.append(jnp.dot(w, v, preferred_element_type=jnp.float32))
            ctx = jnp.concatenate(
                [jnp.concatenate(ctxs[b * HQ_GRP:(b + 1) * HQ_GRP], axis=1)
                 for b in range(B_LOC)], axis=0)
            contrib = jnp.dot(ctx, wo, preferred_element_type=jnp.float32)
            contrib = contrib.reshape(B_LOC, SQ, D_MODEL)
            if first:
                out_ref[...] = contrib
            else:
                out_ref[...] = out_ref[...] + contrib

        compute(my_pos, wq_ref[...], wo_ref[...], first=True)

        for slot, g in ((0, left), (1, right), (2, opp)):
            rdmas[2 * slot].wait_recv()
            rdmas[2 * slot + 1].wait_recv()
            compute(g, wq_comm[slot], wo_comm[slot], first=False)

        for r in rdmas:
            r.wait_send()

    return pl.pallas_call(
        body,
        out_shape=jax.ShapeDtypeStruct((B_LOC, SQ, D_MODEL), jnp.float32),
        in_specs=[pl.BlockSpec(memory_space=pltpu.VMEM)] * 5,
        out_specs=pl.BlockSpec(memory_space=pltpu.VMEM),
        scratch_shapes=[
            pltpu.VMEM((3, D_MODEL, D_QKV), jnp.float32),
            pltpu.VMEM((3, D_QKV, D_MODEL), jnp.float32),
            pltpu.SemaphoreType.DMA((3,)),
            pltpu.SemaphoreType.DMA((3,)),
            pltpu.SemaphoreType.DMA((3,)),
            pltpu.SemaphoreType.DMA((3,)),
        ],
        compiler_params=pltpu.CompilerParams(collective_id=0),
    )(x, Wq, K_own, V_own, Wo)


# device time: 26019 ns/iter; 1.8812x vs baseline; 1.3130x over previous
import jax
import jax.numpy as jnp
from jax import lax
from jax.experimental import pallas as pl
from jax.experimental.pallas import tpu as pltpu

N_DEV = 4
B_LOC = 2
SQ = 128
SKV = 128
HQ = 16
HQ_GRP = 4
DH = 64
D_MODEL = 512
D_QKV = 256

BF16 = jnp.bfloat16
F32 = jnp.float32


def kernel(x, Wq, K_ext, V_ext, Wo):
    my = lax.axis_index("i")
    K_own = jnp.transpose(
        lax.dynamic_slice_in_dim(K_ext, B_LOC * my, B_LOC, axis=0), (0, 2, 1, 3)
    ).astype(BF16)
    V_own = jnp.transpose(
        lax.dynamic_slice_in_dim(V_ext, B_LOC * my, B_LOC, axis=0), (0, 2, 1, 3)
    ).astype(BF16)
    x16 = x.astype(BF16)
    Wq16 = Wq.astype(BF16)
    Wo16 = Wo.astype(BF16)

    def body(x_ref, wq_ref, k_ref, v_ref, wo_ref, out_ref,
             wq_comm, wo_comm, wq_send, wq_recv, wo_send, wo_recv):
        my_pos = lax.axis_index("i")
        left = lax.rem(my_pos + N_DEV - 1, N_DEV)
        right = lax.rem(my_pos + 1, N_DEV)
        opp = lax.rem(my_pos + 2, N_DEV)

        barrier_sem = pltpu.get_barrier_semaphore()
        for nbr in (left, right, opp):
            pl.semaphore_signal(barrier_sem, inc=1, device_id=(nbr,),
                                device_id_type=pl.DeviceIdType.MESH)
        pl.semaphore_wait(barrier_sem, 3)

        rdmas = []
        for slot, tgt in ((0, right), (1, left), (2, opp)):
            for src, comm, ssem, rsem in (
                (wq_ref, wq_comm, wq_send, wq_recv),
                (wo_ref, wo_comm, wo_send, wo_recv),
            ):
                r = pltpu.make_async_remote_copy(
                    src_ref=src, dst_ref=comm.at[slot],
                    send_sem=ssem.at[slot], recv_sem=rsem.at[slot],
                    device_id=(tgt,), device_id_type=pl.DeviceIdType.MESH)
                r.start()
                rdmas.append(r)

        def compute(g, wq, wo, first):
            xm = x_ref[...].reshape(B_LOC * SQ, D_MODEL)
            qm = jnp.dot(xm, wq, preferred_element_type=F32).astype(BF16)
            ctxs = []
            for b in range(B_LOC):
                qb = qm[b * SQ:(b + 1) * SQ]
                for hh in range(HQ_GRP):
                    head = g * HQ_GRP + hh
                    q = qb[:, hh * DH:(hh + 1) * DH]
                    k = k_ref[b, head]
                    v = v_ref[b, head]
                    s = lax.dot_general(
                        q, k, (((1,), (1,)), ((), ())),
                        preferred_element_type=F32) * 0.125
                    m = jnp.max(s, axis=1, keepdims=True)
                    p = jnp.exp(s - m)
                    w = (p / jnp.sum(p, axis=1, keepdims=True)).astype(BF16)
                    ctxs.append(jnp.dot(w, v, preferred_element_type=F32))
            ctx = jnp.concatenate(
                [jnp.concatenate(ctxs[b * HQ_GRP:(b + 1) * HQ_GRP], axis=1)
                 for b in range(B_LOC)], axis=0).astype(BF16)
            contrib = jnp.dot(ctx, wo, preferred_element_type=F32)
            contrib = contrib.reshape(B_LOC, SQ, D_MODEL)
            if first:
                out_ref[...] = contrib
            else:
                out_ref[...] = out_ref[...] + contrib

        compute(my_pos, wq_ref[...], wo_ref[...], first=True)

        for slot, g in ((0, left), (1, right), (2, opp)):
            rdmas[2 * slot].wait_recv()
            rdmas[2 * slot + 1].wait_recv()
            compute(g, wq_comm[slot], wo_comm[slot], first=False)

        for r in rdmas:
            r.wait_send()

    return pl.pallas_call(
        body,
        out_shape=jax.ShapeDtypeStruct((B_LOC, SQ, D_MODEL), F32),
        in_specs=[pl.BlockSpec(memory_space=pltpu.VMEM)] * 5,
        out_specs=pl.BlockSpec(memory_space=pltpu.VMEM),
        scratch_shapes=[
            pltpu.VMEM((3, D_MODEL, D_QKV), BF16),
            pltpu.VMEM((3, D_QKV, D_MODEL), BF16),
            pltpu.SemaphoreType.DMA((3,)),
            pltpu.SemaphoreType.DMA((3,)),
            pltpu.SemaphoreType.DMA((3,)),
            pltpu.SemaphoreType.DMA((3,)),
        ],
        compiler_params=pltpu.CompilerParams(collective_id=0),
    )(x16, Wq16, K_own, V_own, Wo16)
